# baseline (speedup 1.0000x reference)
import functools

import jax
import jax.numpy as jnp
from jax import lax
from jax.experimental import pallas as pl
from jax.experimental.pallas import tpu as pltpu
from jax.experimental.pallas import tpu_sc as plsc

VOCAB = 1000000
BLOCK = 200
EMBED = 64
B = 4096
L = 200
R = B * L
C = 512
IDXW = 128
GPC = C // IDXW
ITEMS = R // C
ITEMS_PER_L = B // C

NC, NS = 2, 16
NW = NC * NS
IPT = ITEMS // NW


def _sc_body(emb_hbm, xt_hbm, pos_hbm, out_hbm, idx_v, rows_v, pos_v, sem):
    wid = lax.axis_index("s") * NC + lax.axis_index("c")
    pltpu.sync_copy(pos_hbm, pos_v)

    @pl.loop(wid * IPT, (wid + 1) * IPT)
    def _(i):
        l = i // ITEMS_PER_L
        pltpu.sync_copy(xt_hbm.at[pl.ds(i * GPC, GPC)], idx_v)
        descs = []
        for j in range(GPC):
            descs.append(pltpu.async_copy(
                emb_hbm.at[idx_v.at[j]],
                rows_v.at[pl.ds(j * IDXW, IDXW)], sem))
        for d in descs:
            d.wait()
        p = [pos_v[l, pl.ds(k * 16, 16)] for k in range(EMBED // 16)]

        @plsc.parallel_loop(0, C, 1, unroll=8)
        def _(r):
            for k in range(EMBED // 16):
                rows_v[r, pl.ds(k * 16, 16)] += p[k]

        pltpu.sync_copy(rows_v, out_hbm.at[pl.ds(i * C, C)])


@jax.jit
def _sc_lookup(emb_table, xt, pos_table):
    mesh = plsc.VectorSubcoreMesh(core_axis_name="c", subcore_axis_name="s")
    return pl.kernel(
        _sc_body,
        out_type=jax.ShapeDtypeStruct((R, EMBED), jnp.float32),
        mesh=mesh,
        compiler_params=pltpu.CompilerParams(use_tc_tiling_on_sc=False),
        scratch_types=[
            pltpu.VMEM((GPC, IDXW), jnp.int32),
            pltpu.VMEM((C, EMBED), jnp.float32),
            pltpu.VMEM((BLOCK, EMBED), jnp.float32),
            pltpu.SemaphoreType.DMA,
        ],
    )(emb_table, xt, pos_table)


def kernel(x, emb_table, pos_table):
    xt = x.T.reshape(R // IDXW, IDXW)
    out = _sc_lookup(emb_table, xt, pos_table)
    return out.reshape(L, B, EMBED)


# speedup vs baseline: 1.0933x; 1.0933x over previous
import jax
import jax.numpy as jnp
from jax import lax
from jax.experimental import pallas as pl
from jax.experimental.pallas import tpu as pltpu
from jax.experimental.pallas import tpu_sc as plsc

VOCAB = 1000000
BLOCK = 200
EMBED = 64
B = 4096
L = 200
R = B * L
C = 512
IDXW = 128
GPC = C // IDXW
ITEMS = R // C
ITEMS_PER_L = B // C

NC, NS = 2, 16
NW = NC * NS
IPT = ITEMS // NW


def _sc_body(emb_hbm, xt_hbm, pos_hbm, out_hbm,
             idx_v, rows_v, pos_v, gsem, osem):
    wid = lax.axis_index("s") * NC + lax.axis_index("c")
    base = wid * IPT
    pltpu.sync_copy(pos_hbm, pos_v)
    pltpu.sync_copy(xt_hbm.at[pl.ds(base * GPC, IPT * GPC)], idx_v)

    def fire(t, b):
        for j in range(GPC):
            pltpu.async_copy(
                emb_hbm.at[idx_v.at[t * GPC + j]],
                rows_v.at[b].at[pl.ds(j * IDXW, IDXW)], gsem[b])

    def wait_gathers(b):
        for j in range(GPC):
            pltpu.make_async_copy(
                emb_hbm.at[idx_v.at[j]],
                rows_v.at[b].at[pl.ds(j * IDXW, IDXW)], gsem[b]).wait()

    def out_desc(t, b):
        return pltpu.make_async_copy(
            rows_v.at[b], out_hbm.at[pl.ds((base + t) * C, C)], osem[b])

    def consume(t, b):
        wait_gathers(b)
        l = (base + t) // ITEMS_PER_L
        p = [pos_v[l, pl.ds(k * 16, 16)] for k in range(EMBED // 16)]

        @plsc.parallel_loop(0, C, 1, unroll=8)
        def _(r):
            for k in range(EMBED // 16):
                rows_v[b, r, pl.ds(k * 16, 16)] += p[k]

        out_desc(t, b).start()

    fire(0, 0)

    @pl.loop(0, IPT, step=2)
    def _(t):
        @pl.when(t > 0)
        def _():
            out_desc(t - 1, 1).wait()
        fire(t + 1, 1)
        consume(t, 0)
        @pl.when(t + 2 < IPT)
        def _():
            out_desc(t, 0).wait()
            fire(t + 2, 0)
        consume(t + 1, 1)

    out_desc(IPT - 2, 0).wait()
    out_desc(IPT - 1, 1).wait()


@jax.jit
def _sc_lookup(emb_table, xt, pos_table):
    mesh = plsc.VectorSubcoreMesh(core_axis_name="c", subcore_axis_name="s")
    return pl.kernel(
        _sc_body,
        out_type=jax.ShapeDtypeStruct((R, EMBED), jnp.float32),
        mesh=mesh,
        compiler_params=pltpu.CompilerParams(use_tc_tiling_on_sc=False),
        scratch_types=[
            pltpu.VMEM((IPT * GPC, IDXW), jnp.int32),
            pltpu.VMEM((2, C, EMBED), jnp.float32),
            pltpu.VMEM((BLOCK, EMBED), jnp.float32),
            [pltpu.SemaphoreType.DMA, pltpu.SemaphoreType.DMA],
            [pltpu.SemaphoreType.DMA, pltpu.SemaphoreType.DMA],
        ],
    )(emb_table, xt, pos_table)


def kernel(x, emb_table, pos_table):
    xt = x.T.reshape(R // IDXW, IDXW)
    out = _sc_lookup(emb_table, xt, pos_table)
    return out.reshape(L, B, EMBED)
